# Initial kernel scaffold; baseline (speedup 1.0000x reference)
import jax
import jax.numpy as jnp
from jax.experimental import pallas as pl


def kernel(voxel_features, voxel_indices, W_hm, b_hm, W_center, b_center, W_center_z, b_center_z, W_dim, b_dim, W_rot, b_rot):
    raise NotImplementedError("write your pallas kernel here")


# speedup vs baseline: 6.8764x; 6.8764x over previous
import functools

import jax
import jax.numpy as jnp
from jax.experimental import pallas as pl
from jax.experimental.pallas import tpu as pltpu

NUM_CLASS = 3
PRE_MAX = 1024
POST_MAX = 500
NMS_THRESH = 0.7
STRIDE = 8
VX = 0.08
VY = 0.08
X0 = -57.6
Y0 = -57.6

_BM = 1000
_IOU_BLK = 128


def _heads_kernel(x_ref, w_ref, b_ref, o_ref):
    y = jax.lax.dot_general(
        x_ref[...], w_ref[...], (((1,), (0,)), ((), ())),
        preferred_element_type=jnp.float32,
    ) + b_ref[...]
    col = jax.lax.broadcasted_iota(jnp.int32, y.shape, 1)
    y_sig = jax.nn.sigmoid(y)
    y_exp = jnp.exp(jnp.clip(y, -4.0, 4.0))
    o_ref[...] = jnp.where(col < 3, y_sig,
                           jnp.where((col >= 6) & (col < 9), y_exp, y))


def _nms_kernel(cand_ref, candt_ref, vidx_ref, vidxt_ref, sc_ref,
                boxes_ref, keep_ref, iou_ref):
    sx = STRIDE * VX
    sy = STRIDE * VY

    xs_c = (vidx_ref[:, 1:2] + cand_ref[:, 3:4]) * sx + X0
    ys_c = (vidx_ref[:, 0:1] + cand_ref[:, 4:5]) * sy + Y0
    d0_c = cand_ref[:, 6:7]
    d1_c = cand_ref[:, 7:8]
    x1_c = xs_c - d0_c * 0.5
    y1_c = ys_c - d1_c * 0.5
    x2_c = xs_c + d0_c * 0.5
    y2_c = ys_c + d1_c * 0.5
    area_c = d0_c * d1_c

    xs_r = (vidxt_ref[1:2, :] + candt_ref[3:4, :]) * sx + X0
    ys_r = (vidxt_ref[0:1, :] + candt_ref[4:5, :]) * sy + Y0
    d0_r = candt_ref[6:7, :]
    d1_r = candt_ref[7:8, :]
    x1_r = xs_r - d0_r * 0.5
    y1_r = ys_r - d1_r * 0.5
    x2_r = xs_r + d0_r * 0.5
    y2_r = ys_r + d1_r * 0.5
    area_r = d0_r * d1_r

    for bi in range(PRE_MAX // _IOU_BLK):
        s = bi * _IOU_BLK
        blk = lambda v: v[s:s + _IOU_BLK, :]
        lt_x = jnp.maximum(blk(x1_c), x1_r)
        lt_y = jnp.maximum(blk(y1_c), y1_r)
        rb_x = jnp.minimum(blk(x2_c), x2_r)
        rb_y = jnp.minimum(blk(y2_c), y2_r)
        w = jnp.maximum(rb_x - lt_x, 0.0)
        h = jnp.maximum(rb_y - lt_y, 0.0)
        inter = w * h
        union = blk(area_c) + area_r - inter
        iou = inter / jnp.maximum(union, 1e-6)
        iou_ref[s:s + _IOU_BLK, :] = (iou > NMS_THRESH).astype(jnp.float32)

    col = jax.lax.broadcasted_iota(jnp.int32, (1, PRE_MAX), 1)

    def body(i, keep):
        row = iou_ref[pl.ds(i, 1), :]
        ki = jnp.sum(jnp.where(col == i, keep, 0.0))
        later = (col > i).astype(jnp.float32)
        return keep * (1.0 - row * later * ki)

    keep = jax.lax.fori_loop(0, PRE_MAX, body,
                             jnp.ones((1, PRE_MAX), jnp.float32),
                             unroll=False)
    keep_ref[...] = keep

    boxes_ref[:, 0:1] = xs_c
    boxes_ref[:, 1:2] = ys_c
    boxes_ref[:, 2:3] = cand_ref[:, 5:6]
    boxes_ref[:, 3:4] = d0_c
    boxes_ref[:, 4:5] = d1_c
    boxes_ref[:, 5:6] = cand_ref[:, 8:9]
    boxes_ref[:, 6:7] = jnp.arctan2(cand_ref[:, 9:10], cand_ref[:, 10:11])
    boxes_ref[:, 7:8] = sc_ref[...]


@functools.partial(jax.jit, static_argnames=("interpret",))
def kernel(voxel_features, voxel_indices, W_hm, b_hm, W_center, b_center,
           W_center_z, b_center_z, W_dim, b_dim, W_rot, b_rot,
           interpret=False):
    n, c = voxel_features.shape
    w_cat = jnp.concatenate(
        [W_hm, W_center, W_center_z, W_dim, W_rot,
         jnp.zeros((c, 5), jnp.float32)], axis=1)
    b_cat = jnp.concatenate(
        [b_hm, b_center, b_center_z, b_dim, b_rot,
         jnp.zeros((5,), jnp.float32)])[None, :]

    heads = pl.pallas_call(
        _heads_kernel,
        grid=(n // _BM,),
        in_specs=[
            pl.BlockSpec((_BM, c), lambda i: (i, 0)),
            pl.BlockSpec((c, 16), lambda i: (0, 0)),
            pl.BlockSpec((1, 16), lambda i: (0, 0)),
        ],
        out_specs=pl.BlockSpec((_BM, 16), lambda i: (i, 0)),
        out_shape=jax.ShapeDtypeStruct((n, 16), jnp.float32),
        interpret=interpret,
    )(voxel_features, w_cat, b_cat)

    hm_flat = heads[:, :NUM_CLASS].reshape(-1)
    top_scores, top_idx = jax.lax.top_k(hm_flat, PRE_MAX)
    vox = top_idx // NUM_CLASS

    cand = heads[vox]
    vidx = voxel_indices[vox].astype(jnp.float32)

    boxes, keep = pl.pallas_call(
        _nms_kernel,
        out_shape=[
            jax.ShapeDtypeStruct((PRE_MAX, 8), jnp.float32),
            jax.ShapeDtypeStruct((1, PRE_MAX), jnp.float32),
        ],
        scratch_shapes=[pltpu.VMEM((PRE_MAX, PRE_MAX), jnp.float32)],
        interpret=interpret,
    )(cand, cand.T, vidx, vidx.T, top_scores[:, None])

    keep_b = keep[0] > 0.5
    order = jnp.argsort(jnp.where(keep_b, 0, 1))[:POST_MAX]
    km = keep[0][order]
    return boxes[order] * km[:, None]


# speedup vs baseline: 6.9176x; 1.0060x over previous
import functools

import jax
import jax.numpy as jnp
from jax.experimental import pallas as pl
from jax.experimental.pallas import tpu as pltpu

NUM_CLASS = 3
PRE_MAX = 1024
POST_MAX = 500
NMS_THRESH = 0.7
STRIDE = 8
VX = 0.08
VY = 0.08
X0 = -57.6
Y0 = -57.6

_BM = 1000
_IOU_BLK = 128


def _heads_kernel(x_ref, w_ref, b_ref, o_ref):
    y = jax.lax.dot_general(
        x_ref[...], w_ref[...], (((1,), (0,)), ((), ())),
        preferred_element_type=jnp.float32,
    ) + b_ref[...]
    col = jax.lax.broadcasted_iota(jnp.int32, y.shape, 1)
    y_sig = jax.nn.sigmoid(y)
    y_exp = jnp.exp(jnp.clip(y, -4.0, 4.0))
    o_ref[...] = jnp.where(col < 3, y_sig,
                           jnp.where((col >= 6) & (col < 9), y_exp, y))


def _nms_kernel(cand_ref, candt_ref, vidx_ref, vidxt_ref, sc_ref,
                boxes_ref, keep_ref, iou_ref):
    sx = STRIDE * VX
    sy = STRIDE * VY

    xs_c = (vidx_ref[:, 1:2] + cand_ref[:, 3:4]) * sx + X0
    ys_c = (vidx_ref[:, 0:1] + cand_ref[:, 4:5]) * sy + Y0
    d0_c = cand_ref[:, 6:7]
    d1_c = cand_ref[:, 7:8]
    x1_c = xs_c - d0_c * 0.5
    y1_c = ys_c - d1_c * 0.5
    x2_c = xs_c + d0_c * 0.5
    y2_c = ys_c + d1_c * 0.5
    area_c = d0_c * d1_c

    xs_r = (vidxt_ref[1:2, :] + candt_ref[3:4, :]) * sx + X0
    ys_r = (vidxt_ref[0:1, :] + candt_ref[4:5, :]) * sy + Y0
    d0_r = candt_ref[6:7, :]
    d1_r = candt_ref[7:8, :]
    x1_r = xs_r - d0_r * 0.5
    y1_r = ys_r - d1_r * 0.5
    x2_r = xs_r + d0_r * 0.5
    y2_r = ys_r + d1_r * 0.5
    area_r = d0_r * d1_r

    for bi in range(PRE_MAX // _IOU_BLK):
        s = bi * _IOU_BLK
        blk = lambda v: v[s:s + _IOU_BLK, :]
        lt_x = jnp.maximum(blk(x1_c), x1_r)
        lt_y = jnp.maximum(blk(y1_c), y1_r)
        rb_x = jnp.minimum(blk(x2_c), x2_r)
        rb_y = jnp.minimum(blk(y2_c), y2_r)
        w = jnp.maximum(rb_x - lt_x, 0.0)
        h = jnp.maximum(rb_y - lt_y, 0.0)
        inter = w * h
        union = blk(area_c) + area_r - inter
        iou = inter / jnp.maximum(union, 1e-6)
        iou_ref[s:s + _IOU_BLK, :] = (iou > NMS_THRESH).astype(jnp.float32)

    col = jax.lax.broadcasted_iota(jnp.int32, (1, PRE_MAX), 1)

    def body(i, keep):
        row = iou_ref[pl.ds(i, 1), :]
        ki = jnp.sum(jnp.where(col == i, keep, 0.0))
        later = (col > i).astype(jnp.float32)
        return keep * (1.0 - row * later * ki)

    keep = jax.lax.fori_loop(0, PRE_MAX, body,
                             jnp.ones((1, PRE_MAX), jnp.float32),
                             unroll=8)
    keep_ref[...] = keep

    boxes_ref[:, 0:1] = xs_c
    boxes_ref[:, 1:2] = ys_c
    boxes_ref[:, 2:3] = cand_ref[:, 5:6]
    boxes_ref[:, 3:4] = d0_c
    boxes_ref[:, 4:5] = d1_c
    boxes_ref[:, 5:6] = cand_ref[:, 8:9]
    boxes_ref[:, 6:7] = jnp.arctan2(cand_ref[:, 9:10], cand_ref[:, 10:11])
    boxes_ref[:, 7:8] = sc_ref[...]


@functools.partial(jax.jit, static_argnames=("interpret",))
def kernel(voxel_features, voxel_indices, W_hm, b_hm, W_center, b_center,
           W_center_z, b_center_z, W_dim, b_dim, W_rot, b_rot,
           interpret=False):
    n, c = voxel_features.shape
    w_cat = jnp.concatenate(
        [W_hm, W_center, W_center_z, W_dim, W_rot,
         jnp.zeros((c, 5), jnp.float32)], axis=1)
    b_cat = jnp.concatenate(
        [b_hm, b_center, b_center_z, b_dim, b_rot,
         jnp.zeros((5,), jnp.float32)])[None, :]

    heads = pl.pallas_call(
        _heads_kernel,
        grid=(n // _BM,),
        in_specs=[
            pl.BlockSpec((_BM, c), lambda i: (i, 0)),
            pl.BlockSpec((c, 16), lambda i: (0, 0)),
            pl.BlockSpec((1, 16), lambda i: (0, 0)),
        ],
        out_specs=pl.BlockSpec((_BM, 16), lambda i: (i, 0)),
        out_shape=jax.ShapeDtypeStruct((n, 16), jnp.float32),
        interpret=interpret,
    )(voxel_features, w_cat, b_cat)

    hm_flat = heads[:, :NUM_CLASS].reshape(-1)
    top_scores, top_idx = jax.lax.top_k(hm_flat, PRE_MAX)
    vox = top_idx // NUM_CLASS

    cand = heads[vox]
    vidx = voxel_indices[vox].astype(jnp.float32)

    boxes, keep = pl.pallas_call(
        _nms_kernel,
        out_shape=[
            jax.ShapeDtypeStruct((PRE_MAX, 8), jnp.float32),
            jax.ShapeDtypeStruct((1, PRE_MAX), jnp.float32),
        ],
        scratch_shapes=[pltpu.VMEM((PRE_MAX, PRE_MAX), jnp.float32)],
        interpret=interpret,
    )(cand, cand.T, vidx, vidx.T, top_scores[:, None])

    keep_b = keep[0] > 0.5
    order = jnp.argsort(jnp.where(keep_b, 0, 1))[:POST_MAX]
    km = keep[0][order]
    return boxes[order] * km[:, None]
